# Initial kernel scaffold; baseline (speedup 1.0000x reference)
import jax
import jax.numpy as jnp
from jax.experimental import pallas as pl


def kernel(pred_ll, target, we, alpha, beta):
    raise NotImplementedError("write your pallas kernel here")


# speedup vs baseline: 1.2725x; 1.2725x over previous
import dataclasses
import functools

import jax
import jax.numpy as jnp
from jax import lax
from jax.experimental import pallas as pl
from jax.experimental.pallas import tpu as pltpu
from jax.experimental.pallas import tpu_sc as plsc

PAD = 0
N_TOK = 4096
VOCAB = 32000
DIM = 1024

TB1 = 256
VB1 = 6400

_NC = 2
_NS = 16
_NW = _NC * _NS
_BPW = N_TOK // _NW
_CHUNK = 64


def _pass1_body(x_ref, tgt_ref, pmax_ref, pos_ref, nll_ref):
    j = pl.program_id(1)
    blk = x_ref[...]
    tcol = tgt_ref[...]
    col = lax.broadcasted_iota(jnp.int32, blk.shape, 1)
    bmax = jnp.max(blk, axis=1, keepdims=True)
    cand = jnp.where(blk == bmax, col, VOCAB)
    bidx = jnp.min(cand, axis=1, keepdims=True) + j * VB1
    teq = col == (tcol - j * VB1)
    tval = jnp.sum(jnp.where(teq, blk, 0.0), axis=1, keepdims=True)

    @pl.when(j == 0)
    def _():
        pmax_ref[...] = bmax
        pos_ref[...] = bidx
        nll_ref[...] = tval

    @pl.when(j != 0)
    def _():
        better = bmax > pmax_ref[...]
        pos_ref[...] = jnp.where(better, bidx, pos_ref[...])
        pmax_ref[...] = jnp.maximum(pmax_ref[...], bmax)
        nll_ref[...] += tval


def _pass1(pred_ll, tgt2d):
    return pl.pallas_call(
        _pass1_body,
        grid=(N_TOK // TB1, VOCAB // VB1),
        in_specs=[
            pl.BlockSpec((TB1, VB1), lambda i, j: (i, j)),
            pl.BlockSpec((TB1, 1), lambda i, j: (i, 0)),
        ],
        out_specs=[
            pl.BlockSpec((TB1, 1), lambda i, j: (i, 0)),
            pl.BlockSpec((TB1, 1), lambda i, j: (i, 0)),
            pl.BlockSpec((TB1, 1), lambda i, j: (i, 0)),
        ],
        out_shape=[
            jax.ShapeDtypeStruct((N_TOK, 1), jnp.float32),
            jax.ShapeDtypeStruct((N_TOK, 1), jnp.int32),
            jax.ShapeDtypeStruct((N_TOK, 1), jnp.float32),
        ],
    )(pred_ll, tgt2d)


def _gather_rows(table, idx2d):
    mesh = plsc.VectorSubcoreMesh(core_axis_name="c", subcore_axis_name="s")
    chunks_per_w = _BPW // _CHUNK

    @functools.partial(
        pl.kernel,
        out_type=jax.ShapeDtypeStruct((N_TOK, DIM), jnp.float32),
        mesh=mesh,
        scratch_types=[
            pltpu.VMEM((_CHUNK,), jnp.int32),
            pltpu.VMEM((_CHUNK, DIM), jnp.float32),
            pltpu.SemaphoreType.DMA,
        ],
    )
    def k(table_hbm, idx_hbm, out_hbm, idx_v, rows_v, sem):
        wid = lax.axis_index("s") * _NC + lax.axis_index("c")
        base = wid * _BPW
        for ci in range(chunks_per_w):
            pltpu.sync_copy(idx_hbm.at[wid * chunks_per_w + ci], idx_v)
            pltpu.async_copy(table_hbm.at[idx_v], rows_v, sem).wait()
            pltpu.sync_copy(
                rows_v, out_hbm.at[pl.ds(base + ci * _CHUNK, _CHUNK)])

    return k(table, idx2d)


def _gather_ab(alpha, beta, tgt):
    mesh = plsc.VectorSubcoreMesh(core_axis_name="c", subcore_axis_name="s")
    cp = pltpu.CompilerParams()
    if "needs_layout_passes" in pltpu.CompilerParams.__dataclass_fields__:
        cp = dataclasses.replace(cp, needs_layout_passes=False)

    @functools.partial(
        pl.kernel,
        compiler_params=cp,
        out_type=[
            jax.ShapeDtypeStruct((N_TOK,), jnp.float32),
            jax.ShapeDtypeStruct((N_TOK,), jnp.float32),
        ],
        mesh=mesh,
        scratch_types=[
            pltpu.VMEM((VOCAB,), jnp.float32),
            pltpu.VMEM((VOCAB,), jnp.float32),
            pltpu.VMEM((_BPW,), jnp.int32),
            pltpu.VMEM((_BPW,), jnp.float32),
            pltpu.VMEM((_BPW,), jnp.float32),
            pltpu.SemaphoreType.DMA,
        ],
    )
    def k(alpha_hbm, beta_hbm, idx_hbm, a_hbm, b_hbm,
          al_v, be_v, idx_v, av, bv, sem):
        wid = lax.axis_index("s") * _NC + lax.axis_index("c")
        base = wid * _BPW
        pltpu.sync_copy(alpha_hbm, al_v)
        pltpu.sync_copy(beta_hbm, be_v)
        pltpu.sync_copy(idx_hbm.at[pl.ds(base, _BPW)], idx_v)
        for t in range(0, _BPW, 16):
            i16 = idx_v[pl.ds(t, 16)]
            av[pl.ds(t, 16)] = plsc.load_gather(al_v, [i16])
            bv[pl.ds(t, 16)] = plsc.load_gather(be_v, [i16])
        pltpu.sync_copy(av, a_hbm.at[pl.ds(base, _BPW)])
        pltpu.sync_copy(bv, b_hbm.at[pl.ds(base, _BPW)])

    return k(alpha, beta, tgt)


TB2 = 512


def _pass2_body(gold_ref, pred_ref, nllv_ref, pmax_ref, a_ref, b_ref,
                tgt_ref, ls_ref, ns_ref):
    i = pl.program_id(0)
    g = gold_ref[...]
    p = pred_ref[...]
    gp = jnp.sum(g * p, axis=1, keepdims=True)
    gg = jnp.sum(g * g, axis=1, keepdims=True)
    pp = jnp.sum(p * p, axis=1, keepdims=True)
    n1 = jnp.maximum(jnp.sqrt(gg), 1e-8)
    n2 = jnp.maximum(jnp.sqrt(pp), 1e-8)
    dist = gp / (n1 * n2)
    z = a_ref[...] * dist + b_ref[...]
    x = 0.5 / (1.0 + jnp.exp(-z))
    nll_loss = -nllv_ref[...]
    pred_loss = -pmax_ref[...]
    m = (tgt_ref[...] != PAD).astype(jnp.float32)
    loss = (x + 0.5) * nll_loss + (0.5 - x) * pred_loss
    ls = jnp.sum(loss * m, axis=0, keepdims=True)
    ns = jnp.sum(nll_loss * m, axis=0, keepdims=True)

    @pl.when(i == 0)
    def _():
        ls_ref[0, 0] = 0.0
        ns_ref[0, 0] = 0.0

    ls_ref[0, 0] += ls[0, 0]
    ns_ref[0, 0] += ns[0, 0]


def _pass2(gold, pred, nllv, pmax, a2d, b2d, tgt2d):
    return pl.pallas_call(
        _pass2_body,
        grid=(N_TOK // TB2,),
        in_specs=[
            pl.BlockSpec((TB2, DIM), lambda i: (i, 0)),
            pl.BlockSpec((TB2, DIM), lambda i: (i, 0)),
            pl.BlockSpec((TB2, 1), lambda i: (i, 0)),
            pl.BlockSpec((TB2, 1), lambda i: (i, 0)),
            pl.BlockSpec((TB2, 1), lambda i: (i, 0)),
            pl.BlockSpec((TB2, 1), lambda i: (i, 0)),
            pl.BlockSpec((TB2, 1), lambda i: (i, 0)),
        ],
        out_specs=[
            pl.BlockSpec(memory_space=pltpu.SMEM),
            pl.BlockSpec(memory_space=pltpu.SMEM),
        ],
        out_shape=[
            jax.ShapeDtypeStruct((1, 1), jnp.float32),
            jax.ShapeDtypeStruct((1, 1), jnp.float32),
        ],
    )(gold, pred, nllv, pmax, a2d, b2d, tgt2d)


def kernel(pred_ll, target, we, alpha, beta):
    tgt = target.reshape(-1).astype(jnp.int32)
    tgt2d = tgt.reshape(N_TOK, 1)
    pmax, pos, nllv = _pass1(pred_ll, tgt2d)
    gold = _gather_rows(we, tgt.reshape(N_TOK // _CHUNK, _CHUNK))
    a1, b1 = _gather_ab(alpha, beta, tgt)
    pred = _gather_rows(we, pos.reshape(N_TOK // _CHUNK, _CHUNK))
    ls, ns = _pass2(gold, pred, nllv, pmax,
                    a1.reshape(N_TOK, 1), b1.reshape(N_TOK, 1), tgt2d)
    return ls.reshape(()), ns.reshape(())


# speedup vs baseline: 1.3050x; 1.0256x over previous
import dataclasses
import functools

import jax
import jax.numpy as jnp
from jax import lax
from jax.experimental import pallas as pl
from jax.experimental.pallas import tpu as pltpu
from jax.experimental.pallas import tpu_sc as plsc

PAD = 0
N_TOK = 4096
VOCAB = 32000
DIM = 1024

TB1 = 64

_NC = 2
_NS = 16
_NW = _NC * _NS
_BPW = N_TOK // _NW
_CHUNK = 64


def _pass1_body(x_ref, tgt_ref, pmax_ref, pos_ref, nll_ref):
    blk = x_ref[...]
    tcol = tgt_ref[...].astype(jnp.float32)
    bmax = jnp.max(blk, axis=1, keepdims=True)
    ncol = lax.broadcasted_iota(jnp.int32, blk.shape, 1).astype(jnp.float32)
    nidx = jnp.max(jnp.where(blk == bmax, -ncol, -3.4e38),
                   axis=1, keepdims=True)
    tval = jnp.max(jnp.where(ncol == tcol, blk, -3.4e38),
                   axis=1, keepdims=True)
    pmax_ref[...] = bmax
    pos_ref[...] = (-nidx).astype(jnp.int32)
    nll_ref[...] = tval


def _pass1(pred_ll, tgt2d):
    return pl.pallas_call(
        _pass1_body,
        grid=(N_TOK // TB1,),
        in_specs=[
            pl.BlockSpec((TB1, VOCAB), lambda i: (i, 0)),
            pl.BlockSpec((TB1, 1), lambda i: (i, 0)),
        ],
        out_specs=[
            pl.BlockSpec((TB1, 1), lambda i: (i, 0)),
            pl.BlockSpec((TB1, 1), lambda i: (i, 0)),
            pl.BlockSpec((TB1, 1), lambda i: (i, 0)),
        ],
        out_shape=[
            jax.ShapeDtypeStruct((N_TOK, 1), jnp.float32),
            jax.ShapeDtypeStruct((N_TOK, 1), jnp.int32),
            jax.ShapeDtypeStruct((N_TOK, 1), jnp.float32),
        ],
    )(pred_ll, tgt2d)


def _gather_rows(table, idx2d, dim):
    mesh = plsc.VectorSubcoreMesh(core_axis_name="c", subcore_axis_name="s")
    chunks_per_w = _BPW // _CHUNK

    @functools.partial(
        pl.kernel,
        out_type=jax.ShapeDtypeStruct((N_TOK, dim), jnp.float32),
        mesh=mesh,
        scratch_types=[
            pltpu.VMEM((_CHUNK,), jnp.int32),
            pltpu.VMEM((_CHUNK, dim), jnp.float32),
            pltpu.SemaphoreType.DMA,
        ],
    )
    def k(table_hbm, idx_hbm, out_hbm, idx_v, rows_v, sem):
        wid = lax.axis_index("s") * _NC + lax.axis_index("c")
        base = wid * _BPW
        for ci in range(chunks_per_w):
            pltpu.sync_copy(idx_hbm.at[wid * chunks_per_w + ci], idx_v)
            pltpu.async_copy(table_hbm.at[idx_v], rows_v, sem).wait()
            pltpu.sync_copy(
                rows_v, out_hbm.at[pl.ds(base + ci * _CHUNK, _CHUNK)])

    return k(table, idx2d)


def _gather_ab(alpha, beta, tgt):
    mesh = plsc.VectorSubcoreMesh(core_axis_name="c", subcore_axis_name="s")
    cp = pltpu.CompilerParams()
    if "needs_layout_passes" in pltpu.CompilerParams.__dataclass_fields__:
        cp = dataclasses.replace(cp, needs_layout_passes=False)

    @functools.partial(
        pl.kernel,
        compiler_params=cp,
        out_type=[
            jax.ShapeDtypeStruct((N_TOK,), jnp.float32),
            jax.ShapeDtypeStruct((N_TOK,), jnp.float32),
        ],
        mesh=mesh,
        scratch_types=[
            pltpu.VMEM((VOCAB,), jnp.float32),
            pltpu.VMEM((VOCAB,), jnp.float32),
            pltpu.VMEM((_BPW,), jnp.int32),
            pltpu.VMEM((_BPW,), jnp.float32),
            pltpu.VMEM((_BPW,), jnp.float32),
            pltpu.SemaphoreType.DMA,
        ],
    )
    def k(alpha_hbm, beta_hbm, idx_hbm, a_hbm, b_hbm,
          al_v, be_v, idx_v, av, bv, sem):
        wid = lax.axis_index("s") * _NC + lax.axis_index("c")
        base = wid * _BPW
        pltpu.sync_copy(alpha_hbm, al_v)
        pltpu.sync_copy(beta_hbm, be_v)
        pltpu.sync_copy(idx_hbm.at[pl.ds(base, _BPW)], idx_v)
        for t in range(0, _BPW, 16):
            i16 = idx_v[pl.ds(t, 16)]
            av[pl.ds(t, 16)] = plsc.load_gather(al_v, [i16])
            bv[pl.ds(t, 16)] = plsc.load_gather(be_v, [i16])
        pltpu.sync_copy(av, a_hbm.at[pl.ds(base, _BPW)])
        pltpu.sync_copy(bv, b_hbm.at[pl.ds(base, _BPW)])

    return k(alpha, beta, tgt)


TB2 = 512


def _pass2_body(gold_ref, pred_ref, nllv_ref, pmax_ref, a_ref, b_ref,
                tgt_ref, ls_ref, ns_ref):
    i = pl.program_id(0)
    g = gold_ref[...]
    p = pred_ref[...]
    gp = jnp.sum(g * p, axis=1, keepdims=True)
    gg = jnp.sum(g * g, axis=1, keepdims=True)
    pp = jnp.sum(p * p, axis=1, keepdims=True)
    n1 = jnp.maximum(jnp.sqrt(gg), 1e-8)
    n2 = jnp.maximum(jnp.sqrt(pp), 1e-8)
    dist = gp / (n1 * n2)
    z = a_ref[...] * dist + b_ref[...]
    x = 0.5 / (1.0 + jnp.exp(-z))
    nll_loss = -nllv_ref[...]
    pred_loss = -pmax_ref[...]
    m = (tgt_ref[...] != PAD).astype(jnp.float32)
    loss = (x + 0.5) * nll_loss + (0.5 - x) * pred_loss
    ls = jnp.sum(loss * m, axis=0, keepdims=True)
    ns = jnp.sum(nll_loss * m, axis=0, keepdims=True)

    @pl.when(i == 0)
    def _():
        ls_ref[0, 0] = 0.0
        ns_ref[0, 0] = 0.0

    ls_ref[0, 0] += ls[0, 0]
    ns_ref[0, 0] += ns[0, 0]


def _pass2(gold, pred, nllv, pmax, a2d, b2d, tgt2d):
    return pl.pallas_call(
        _pass2_body,
        grid=(N_TOK // TB2,),
        in_specs=[
            pl.BlockSpec((TB2, DIM), lambda i: (i, 0)),
            pl.BlockSpec((TB2, DIM), lambda i: (i, 0)),
            pl.BlockSpec((TB2, 1), lambda i: (i, 0)),
            pl.BlockSpec((TB2, 1), lambda i: (i, 0)),
            pl.BlockSpec((TB2, 1), lambda i: (i, 0)),
            pl.BlockSpec((TB2, 1), lambda i: (i, 0)),
            pl.BlockSpec((TB2, 1), lambda i: (i, 0)),
        ],
        out_specs=[
            pl.BlockSpec(memory_space=pltpu.SMEM),
            pl.BlockSpec(memory_space=pltpu.SMEM),
        ],
        out_shape=[
            jax.ShapeDtypeStruct((1, 1), jnp.float32),
            jax.ShapeDtypeStruct((1, 1), jnp.float32),
        ],
    )(gold, pred, nllv, pmax, a2d, b2d, tgt2d)


def kernel(pred_ll, target, we, alpha, beta):
    tgt = target.reshape(-1).astype(jnp.int32)
    tgt2d = tgt.reshape(N_TOK, 1)
    pmax, pos, nllv = _pass1(pred_ll, tgt2d)
    gold = _gather_rows(we, tgt.reshape(N_TOK // _CHUNK, _CHUNK), DIM)
    a1, b1 = _gather_ab(alpha, beta, tgt)
    pred = _gather_rows(we, pos.reshape(N_TOK // _CHUNK, _CHUNK), DIM)
    ls, ns = _pass2(gold, pred, nllv, pmax,
                    a1.reshape(N_TOK, 1), b1.reshape(N_TOK, 1), tgt2d)
    return ls.reshape(()), ns.reshape(())


# speedup vs baseline: 1.4145x; 1.0839x over previous
import dataclasses
import functools

import jax
import jax.numpy as jnp
from jax import lax
from jax.experimental import pallas as pl
from jax.experimental.pallas import tpu as pltpu
from jax.experimental.pallas import tpu_sc as plsc

PAD = 0
N_TOK = 4096
VOCAB = 32000
DIM = 1024

TB1 = 128

_NC = 2
_NS = 16
_NW = _NC * _NS
_BPW = N_TOK // _NW
_CHUNK = 64


def _pass1_body(x_ref, tgt_ref, pmax_ref, pos_ref, nll_ref):
    blk = x_ref[...]
    tcol = tgt_ref[...].astype(jnp.float32)
    bmax = jnp.max(blk, axis=1, keepdims=True)
    ncol = lax.broadcasted_iota(jnp.int32, blk.shape, 1).astype(jnp.float32)
    nidx = jnp.max(jnp.where(blk == bmax, -ncol, -3.4e38),
                   axis=1, keepdims=True)
    tval = jnp.max(jnp.where(ncol == tcol, blk, -3.4e38),
                   axis=1, keepdims=True)
    pmax_ref[...] = bmax
    pos_ref[...] = (-nidx).astype(jnp.int32)
    nll_ref[...] = tval


def _pass1(pred_ll, tgt2d):
    return pl.pallas_call(
        _pass1_body,
        grid=(N_TOK // TB1,),
        in_specs=[
            pl.BlockSpec((TB1, VOCAB), lambda i: (i, 0)),
            pl.BlockSpec((TB1, 1), lambda i: (i, 0)),
        ],
        out_specs=[
            pl.BlockSpec((TB1, 1), lambda i: (i, 0)),
            pl.BlockSpec((TB1, 1), lambda i: (i, 0)),
            pl.BlockSpec((TB1, 1), lambda i: (i, 0)),
        ],
        out_shape=[
            jax.ShapeDtypeStruct((N_TOK, 1), jnp.float32),
            jax.ShapeDtypeStruct((N_TOK, 1), jnp.int32),
            jax.ShapeDtypeStruct((N_TOK, 1), jnp.float32),
        ],
    )(pred_ll, tgt2d)


def _gather_rows(table, idx2d, dim):
    mesh = plsc.VectorSubcoreMesh(core_axis_name="c", subcore_axis_name="s")
    chunks_per_w = _BPW // _CHUNK

    @functools.partial(
        pl.kernel,
        out_type=jax.ShapeDtypeStruct((N_TOK, dim), jnp.float32),
        mesh=mesh,
        scratch_types=[
            pltpu.VMEM((_CHUNK,), jnp.int32),
            pltpu.VMEM((_CHUNK, dim), jnp.float32),
            pltpu.SemaphoreType.DMA,
        ],
    )
    def k(table_hbm, idx_hbm, out_hbm, idx_v, rows_v, sem):
        wid = lax.axis_index("s") * _NC + lax.axis_index("c")
        base = wid * _BPW
        for ci in range(chunks_per_w):
            pltpu.sync_copy(idx_hbm.at[wid * chunks_per_w + ci], idx_v)
            pltpu.async_copy(table_hbm.at[idx_v], rows_v, sem).wait()
            pltpu.sync_copy(
                rows_v, out_hbm.at[pl.ds(base + ci * _CHUNK, _CHUNK)])

    return k(table, idx2d)


def _gather_ab(alpha, beta, tgt):
    mesh = plsc.VectorSubcoreMesh(core_axis_name="c", subcore_axis_name="s")
    cp = pltpu.CompilerParams()
    if "needs_layout_passes" in pltpu.CompilerParams.__dataclass_fields__:
        cp = dataclasses.replace(cp, needs_layout_passes=False)

    @functools.partial(
        pl.kernel,
        compiler_params=cp,
        out_type=[
            jax.ShapeDtypeStruct((N_TOK,), jnp.float32),
            jax.ShapeDtypeStruct((N_TOK,), jnp.float32),
        ],
        mesh=mesh,
        scratch_types=[
            pltpu.VMEM((VOCAB,), jnp.float32),
            pltpu.VMEM((VOCAB,), jnp.float32),
            pltpu.VMEM((_BPW,), jnp.int32),
            pltpu.VMEM((_BPW,), jnp.float32),
            pltpu.VMEM((_BPW,), jnp.float32),
            pltpu.SemaphoreType.DMA,
        ],
    )
    def k(alpha_hbm, beta_hbm, idx_hbm, a_hbm, b_hbm,
          al_v, be_v, idx_v, av, bv, sem):
        wid = lax.axis_index("s") * _NC + lax.axis_index("c")
        base = wid * _BPW
        pltpu.sync_copy(alpha_hbm, al_v)
        pltpu.sync_copy(beta_hbm, be_v)
        pltpu.sync_copy(idx_hbm.at[pl.ds(base, _BPW)], idx_v)
        for t in range(0, _BPW, 16):
            i16 = idx_v[pl.ds(t, 16)]
            av[pl.ds(t, 16)] = plsc.load_gather(al_v, [i16])
            bv[pl.ds(t, 16)] = plsc.load_gather(be_v, [i16])
        pltpu.sync_copy(av, a_hbm.at[pl.ds(base, _BPW)])
        pltpu.sync_copy(bv, b_hbm.at[pl.ds(base, _BPW)])

    return k(alpha, beta, tgt)


TB2 = 512


def _pass2_body(gold_ref, pred_ref, nllv_ref, pmax_ref, a_ref, b_ref,
                tgt_ref, ls_ref, ns_ref):
    i = pl.program_id(0)
    g = gold_ref[...]
    p = pred_ref[...]
    gp = jnp.sum(g * p, axis=1, keepdims=True)
    gg = jnp.sum(g * g, axis=1, keepdims=True)
    pp = jnp.sum(p * p, axis=1, keepdims=True)
    n1 = jnp.maximum(jnp.sqrt(gg), 1e-8)
    n2 = jnp.maximum(jnp.sqrt(pp), 1e-8)
    dist = gp / (n1 * n2)
    z = a_ref[...] * dist + b_ref[...]
    x = 0.5 / (1.0 + jnp.exp(-z))
    nll_loss = -nllv_ref[...]
    pred_loss = -pmax_ref[...]
    m = (tgt_ref[...] != PAD).astype(jnp.float32)
    loss = (x + 0.5) * nll_loss + (0.5 - x) * pred_loss
    ls = jnp.sum(loss * m, axis=0, keepdims=True)
    ns = jnp.sum(nll_loss * m, axis=0, keepdims=True)

    @pl.when(i == 0)
    def _():
        ls_ref[0, 0] = 0.0
        ns_ref[0, 0] = 0.0

    ls_ref[0, 0] += ls[0, 0]
    ns_ref[0, 0] += ns[0, 0]


def _pass2(gold, pred, nllv, pmax, a2d, b2d, tgt2d):
    return pl.pallas_call(
        _pass2_body,
        grid=(N_TOK // TB2,),
        in_specs=[
            pl.BlockSpec((TB2, DIM), lambda i: (i, 0)),
            pl.BlockSpec((TB2, DIM), lambda i: (i, 0)),
            pl.BlockSpec((TB2, 1), lambda i: (i, 0)),
            pl.BlockSpec((TB2, 1), lambda i: (i, 0)),
            pl.BlockSpec((TB2, 1), lambda i: (i, 0)),
            pl.BlockSpec((TB2, 1), lambda i: (i, 0)),
            pl.BlockSpec((TB2, 1), lambda i: (i, 0)),
        ],
        out_specs=[
            pl.BlockSpec(memory_space=pltpu.SMEM),
            pl.BlockSpec(memory_space=pltpu.SMEM),
        ],
        out_shape=[
            jax.ShapeDtypeStruct((1, 1), jnp.float32),
            jax.ShapeDtypeStruct((1, 1), jnp.float32),
        ],
    )(gold, pred, nllv, pmax, a2d, b2d, tgt2d)


def kernel(pred_ll, target, we, alpha, beta):
    tgt = target.reshape(-1).astype(jnp.int32)
    tgt2d = tgt.reshape(N_TOK, 1)
    pmax, pos, nllv = _pass1(pred_ll, tgt2d)
    gold = _gather_rows(we, tgt.reshape(N_TOK // _CHUNK, _CHUNK), DIM)
    a1, b1 = _gather_ab(alpha, beta, tgt)
    pred = _gather_rows(we, pos.reshape(N_TOK // _CHUNK, _CHUNK), DIM)
    ls, ns = _pass2(gold, pred, nllv, pmax,
                    a1.reshape(N_TOK, 1), b1.reshape(N_TOK, 1), tgt2d)
    return ls.reshape(()), ns.reshape(())
